# baseline (speedup 1.0000x reference)
import functools

import jax
import jax.numpy as jnp
from jax.experimental import pallas as pl


def _moe_block(x_ref, wg_ref, bg_ref, w1_ref, b1_ref, w2_ref, b2_ref, o_ref,
               *, num_experts, expert_size):
    E, F = num_experts, expert_size
    xb = x_ref[...]
    logits = jnp.dot(xb, wg_ref[...], preferred_element_type=jnp.float32)
    logits = logits + bg_ref[...]
    m = jnp.max(logits, axis=-1, keepdims=True)
    p = jnp.exp(logits - m)
    p = p / jnp.sum(p, axis=-1, keepdims=True)

    i1 = jnp.argmax(p, axis=-1)[:, None]
    top1 = jnp.max(p, axis=-1, keepdims=True)
    cols = jax.lax.broadcasted_iota(jnp.int32, p.shape, 1)
    p2 = jnp.where(cols == i1, -jnp.inf, p)
    i2 = jnp.argmax(p2, axis=-1)[:, None]
    top2 = jnp.max(p2, axis=-1, keepdims=True)
    denom = top1 + top2
    c = (jnp.where(cols == i1, top1, 0.0) + jnp.where(cols == i2, top2, 0.0)) / denom

    h = jnp.dot(xb.astype(jnp.bfloat16), w1_ref[...],
                preferred_element_type=jnp.float32) + b1_ref[...]
    h = jnp.maximum(h, 0.0)
    hw = (h.reshape(-1, E, F) * c[:, :, None]).reshape(-1, E * F)
    y = jnp.dot(hw.astype(jnp.bfloat16), w2_ref[...],
                preferred_element_type=jnp.float32)
    y = y + jnp.dot(c, b2_ref[...], preferred_element_type=jnp.float32)
    o_ref[...] = y


def kernel(x, Wg, bg, W1, b1, W2, b2):
    B, S, H = x.shape
    E, _, F = W1.shape
    N = B * S
    xf = x.reshape(N, H)
    M = 512
    grid = (N // M,)

    W1a = W1.transpose(1, 0, 2).reshape(H, E * F).astype(jnp.bfloat16)
    W2a = W2.reshape(E * F, H).astype(jnp.bfloat16)
    b1a = b1.reshape(1, E * F)

    out = pl.pallas_call(
        functools.partial(_moe_block, num_experts=E, expert_size=F),
        grid=grid,
        in_specs=[
            pl.BlockSpec((M, H), lambda i: (i, 0)),
            pl.BlockSpec((H, E), lambda i: (0, 0)),
            pl.BlockSpec((1, E), lambda i: (0, 0)),
            pl.BlockSpec((H, E * F), lambda i: (0, 0)),
            pl.BlockSpec((1, E * F), lambda i: (0, 0)),
            pl.BlockSpec((E * F, H), lambda i: (0, 0)),
            pl.BlockSpec((E, H), lambda i: (0, 0)),
        ],
        out_specs=pl.BlockSpec((M, H), lambda i: (i, 0)),
        out_shape=jax.ShapeDtypeStruct((N, H), jnp.float32),
    )(xf, Wg, bg.reshape(1, E), W1a, b1a, W2a, b2)
    return out.reshape(B, S, H)


# speedup vs baseline: 1.2906x; 1.2906x over previous
import functools

import jax
import jax.numpy as jnp
from jax.experimental import pallas as pl


def _moe_block(x_ref, wg_ref, bg_ref, w1_ref, b1_ref, w2_ref, b2_ref,
               exp_ref, o_ref, *, num_experts, expert_size):
    E, F = num_experts, expert_size
    xb = x_ref[...]
    logits = jnp.dot(xb, wg_ref[...], preferred_element_type=jnp.float32)
    logits = logits + bg_ref[...]
    m = jnp.max(logits, axis=-1, keepdims=True)
    p = jnp.exp(logits - m)
    p = p / jnp.sum(p, axis=-1, keepdims=True)

    i1 = jnp.argmax(p, axis=-1)[:, None]
    top1 = jnp.max(p, axis=-1, keepdims=True)
    cols = jax.lax.broadcasted_iota(jnp.int32, p.shape, 1)
    p2 = jnp.where(cols == i1, -jnp.inf, p)
    i2 = jnp.argmax(p2, axis=-1)[:, None]
    top2 = jnp.max(p2, axis=-1, keepdims=True)
    denom = top1 + top2
    c = (jnp.where(cols == i1, top1, 0.0) + jnp.where(cols == i2, top2, 0.0)) / denom

    h = jnp.dot(xb.astype(jnp.bfloat16), w1_ref[...],
                preferred_element_type=jnp.float32) + b1_ref[...]
    h = jnp.maximum(h, 0.0)
    cw = jnp.dot(c, exp_ref[...], preferred_element_type=jnp.float32)
    y = jnp.dot((h * cw).astype(jnp.bfloat16), w2_ref[...],
                preferred_element_type=jnp.float32)
    y = y + jnp.dot(c, b2_ref[...], preferred_element_type=jnp.float32)
    o_ref[...] = y


def kernel(x, Wg, bg, W1, b1, W2, b2):
    B, S, H = x.shape
    E, _, F = W1.shape
    N = B * S
    xf = x.reshape(N, H)
    M = 512
    grid = (N // M,)

    W1a = W1.transpose(1, 0, 2).reshape(H, E * F).astype(jnp.bfloat16)
    W2a = W2.reshape(E * F, H).astype(jnp.bfloat16)
    b1a = b1.reshape(1, E * F)
    expand = jnp.repeat(jnp.eye(E, dtype=jnp.float32), F, axis=1)

    out = pl.pallas_call(
        functools.partial(_moe_block, num_experts=E, expert_size=F),
        grid=grid,
        in_specs=[
            pl.BlockSpec((M, H), lambda i: (i, 0)),
            pl.BlockSpec((H, E), lambda i: (0, 0)),
            pl.BlockSpec((1, E), lambda i: (0, 0)),
            pl.BlockSpec((H, E * F), lambda i: (0, 0)),
            pl.BlockSpec((1, E * F), lambda i: (0, 0)),
            pl.BlockSpec((E * F, H), lambda i: (0, 0)),
            pl.BlockSpec((E, H), lambda i: (0, 0)),
            pl.BlockSpec((E, E * F), lambda i: (0, 0)),
        ],
        out_specs=pl.BlockSpec((M, H), lambda i: (i, 0)),
        out_shape=jax.ShapeDtypeStruct((N, H), jnp.float32),
    )(xf, Wg, bg.reshape(1, E), W1a, b1a, W2a, b2, expand)
    return out.reshape(B, S, H)
